# baseline (speedup 1.0000x reference)
import jax
import jax.numpy as jnp
from jax.experimental import pallas as pl
from jax.experimental.pallas import tpu as pltpu

N = 19
TB = 32
TBL = 4


def _blockdiag(lap_stack):
    tiled = jnp.concatenate([lap_stack] * TBL, axis=1)
    row_g = jax.lax.broadcasted_iota(jnp.int32, tiled.shape, 0) // N
    col_g = jax.lax.broadcasted_iota(jnp.int32, tiled.shape, 1) // N
    return jnp.where(row_g == col_g, tiled, 0.0)


def _cheb1_kernel(x_ref, a_ref, o_ref):
    nbands = x_ref.shape[0]
    nsub = TB // TBL
    for band in range(nbands):
        x0 = x_ref[band]
        a = a_ref[band]
        x1s, x2s = [], []
        for s in range(nsub):
            bd = _blockdiag(a[s * TBL * N:(s + 1) * TBL * N])
            x0s = x0[s * TBL * N:(s + 1) * TBL * N]
            x1 = jnp.dot(bd, x0s, preferred_element_type=jnp.float32)
            x2 = 2.0 * jnp.dot(bd, x1, preferred_element_type=jnp.float32) - x0s
            x1s.append(x1)
            x2s.append(x2)
        xc = jnp.concatenate(
            [x0, jnp.concatenate(x1s, 0), jnp.concatenate(x2s, 0)], axis=-1)
        o_ref[band] = xc


def _stack_kernel(xc_ref, a_ref, wi_ref, bi_ref, wh_ref, bh_ref, wo_ref,
                  bo_ref, wtap_ref, bconv_ref, o_ref):
    nbands = xc_ref.shape[0]
    nh = wh_ref.shape[1] // nbands
    nsub = TB // TBL
    lout = o_ref.shape[-1]

    band_outs = []
    for band in range(nbands):
        a = a_ref[band]
        bds = [_blockdiag(a[s * TBL * N:(s + 1) * TBL * N])
               for s in range(nsub)]

        def cheb(h, w, b, relu):
            x1s, x2s = [], []
            for s in range(nsub):
                hs = h[s * TBL * N:(s + 1) * TBL * N]
                x1 = jnp.dot(bds[s], hs, preferred_element_type=jnp.float32)
                x2 = 2.0 * jnp.dot(bds[s], x1,
                                   preferred_element_type=jnp.float32) - hs
                x1s.append(x1)
                x2s.append(x2)
            xc = jnp.concatenate(
                [h, jnp.concatenate(x1s, 0), jnp.concatenate(x2s, 0)], -1)
            y = jnp.dot(xc, w, preferred_element_type=jnp.float32) + b
            return jnp.maximum(y, 0.0) if relu else y

        h = jnp.maximum(
            jnp.dot(xc_ref[band], wi_ref[0, band],
                    preferred_element_type=jnp.float32) + bi_ref[0, band], 0.0)
        for l in range(nh):
            h = cheb(h, wh_ref[0, band * nh + l], bh_ref[0, band * nh + l],
                     True)
        h = cheb(h, wo_ref[0, band], bo_ref[0, band], False)
        band_outs.append(h)

    g = jnp.concatenate(band_outs, axis=-1)
    taps = []
    for k in range(3):
        p = g * wtap_ref[0, k]
        taps.append(jnp.sum(p.reshape(TB, N, g.shape[-1]), axis=1))
    feats = sum(taps[k][:, k:k + lout] for k in range(3)) + bconv_ref[0]
    o_ref[0] = feats


def _head_kernel(f_ref, gamma_ref, beta_ref, fcw_ref, fcb_ref, o_ref):
    feats = f_ref[0]
    mean = jnp.mean(feats, axis=0, keepdims=True)
    var = jnp.mean((feats - mean) ** 2, axis=0, keepdims=True)
    xn = (feats - mean) * jax.lax.rsqrt(var + 1e-5)
    xn = xn * gamma_ref[0] + beta_ref[0]
    o_ref[0] = (jnp.dot(xn, fcw_ref[0], preferred_element_type=jnp.float32)
                + fcb_ref[0])


def kernel(x, A, wi, bi, wh, bh, wo, bo, wtap, bconv, gamma, beta, fcw, fcb):
    B, _, fin, nbands = x.shape
    M = wi.shape[0]
    H = wi.shape[-1]
    nh2 = wh.shape[1]
    co = wo.shape[-1]
    lout = gamma.shape[-1]
    C = fcw.shape[-1]
    nbt = B // TB

    xb = jnp.transpose(x, (3, 0, 1, 2)).reshape(nbands, B * N, fin)
    ab = jnp.transpose(A, (1, 0, 2, 3)).reshape(nbands, B * N, N)
    wtap_t = jnp.tile(wtap, (1, 1, TB, 1))

    xc1 = pl.pallas_call(
        _cheb1_kernel,
        out_shape=jax.ShapeDtypeStruct((nbands, B * N, 3 * fin), jnp.float32),
        grid=(nbt,),
        in_specs=[
            pl.BlockSpec((nbands, TB * N, fin), lambda i: (0, i, 0)),
            pl.BlockSpec((nbands, TB * N, N), lambda i: (0, i, 0)),
        ],
        out_specs=pl.BlockSpec((nbands, TB * N, 3 * fin), lambda i: (0, i, 0)),
        compiler_params=pltpu.CompilerParams(
            dimension_semantics=("parallel",)),
    )(xb, ab)

    feats = pl.pallas_call(
        _stack_kernel,
        out_shape=jax.ShapeDtypeStruct((M, B, lout), jnp.float32),
        grid=(nbt, M),
        in_specs=[
            pl.BlockSpec((nbands, TB * N, 3 * fin), lambda bt, m: (0, bt, 0)),
            pl.BlockSpec((nbands, TB * N, N), lambda bt, m: (0, bt, 0)),
            pl.BlockSpec((1, nbands, 3 * fin, H), lambda bt, m: (m, 0, 0, 0)),
            pl.BlockSpec((1, nbands, 1, H), lambda bt, m: (m, 0, 0, 0)),
            pl.BlockSpec((1, nh2, 3 * H, H), lambda bt, m: (m, 0, 0, 0)),
            pl.BlockSpec((1, nh2, 1, H), lambda bt, m: (m, 0, 0, 0)),
            pl.BlockSpec((1, nbands, 3 * H, co), lambda bt, m: (m, 0, 0, 0)),
            pl.BlockSpec((1, nbands, 1, co), lambda bt, m: (m, 0, 0, 0)),
            pl.BlockSpec((1, 3, TB * N, 1), lambda bt, m: (m, 0, 0, 0)),
            pl.BlockSpec((1, 1, 1), lambda bt, m: (m, 0, 0)),
        ],
        out_specs=pl.BlockSpec((1, TB, lout), lambda bt, m: (m, bt, 0)),
        compiler_params=pltpu.CompilerParams(
            dimension_semantics=("parallel", "arbitrary")),
    )(xc1, ab, wi, bi, wh, bh, wo, bo, wtap_t, bconv)

    logits = pl.pallas_call(
        _head_kernel,
        out_shape=jax.ShapeDtypeStruct((M, B, C), jnp.float32),
        grid=(M,),
        in_specs=[
            pl.BlockSpec((1, B, lout), lambda m: (m, 0, 0)),
            pl.BlockSpec((1, 1, lout), lambda m: (m, 0, 0)),
            pl.BlockSpec((1, 1, lout), lambda m: (m, 0, 0)),
            pl.BlockSpec((1, lout, C), lambda m: (m, 0, 0)),
            pl.BlockSpec((1, 1, C), lambda m: (m, 0, 0)),
        ],
        out_specs=pl.BlockSpec((1, B, C), lambda m: (m, 0, 0)),
        compiler_params=pltpu.CompilerParams(
            dimension_semantics=("parallel",)),
    )(feats, gamma, beta, fcw, fcb)

    return jnp.mean(logits, axis=0)


# speedup vs baseline: 1.2055x; 1.2055x over previous
import jax
import jax.numpy as jnp
from jax.experimental import pallas as pl
from jax.experimental.pallas import tpu as pltpu

N = 19
TB = 32
TBL = 8


def _mm(a, b):
    return jnp.dot(a.astype(jnp.bfloat16), b.astype(jnp.bfloat16),
                   preferred_element_type=jnp.float32)


def _blockdiag(lap_stack):
    tiled = jnp.concatenate([lap_stack] * TBL, axis=1)
    row_g = jax.lax.broadcasted_iota(jnp.int32, tiled.shape, 0) // N
    col_g = jax.lax.broadcasted_iota(jnp.int32, tiled.shape, 1) // N
    return jnp.where(row_g == col_g, tiled, 0.0)


def _cheb1_kernel(x_ref, a_ref, o_ref):
    nbands = x_ref.shape[0]
    nsub = TB // TBL
    for band in range(nbands):
        x0 = x_ref[band]
        a = a_ref[band]
        x1s, x2s = [], []
        for s in range(nsub):
            bd = _blockdiag(a[s * TBL * N:(s + 1) * TBL * N])
            x0s = x0[s * TBL * N:(s + 1) * TBL * N]
            x1 = _mm(bd, x0s)
            x2 = 2.0 * _mm(bd, x1) - x0s
            x1s.append(x1)
            x2s.append(x2)
        xc = jnp.concatenate(
            [x0, jnp.concatenate(x1s, 0), jnp.concatenate(x2s, 0)], axis=-1)
        o_ref[band] = xc


def _stack_kernel(xc_ref, a_ref, wi_ref, bi_ref, wh_ref, bh_ref, wo_ref,
                  bo_ref, wtap_ref, bconv_ref, o_ref):
    nbands = xc_ref.shape[0]
    nh = wh_ref.shape[1] // nbands
    nsub = TB // TBL
    lout = o_ref.shape[-1]

    band_outs = []
    for band in range(nbands):
        a = a_ref[band]
        bds = [_blockdiag(a[s * TBL * N:(s + 1) * TBL * N])
               for s in range(nsub)]

        def cheb(h, w, b, relu):
            x1s, x2s = [], []
            for s in range(nsub):
                hs = h[s * TBL * N:(s + 1) * TBL * N]
                x1 = _mm(bds[s], hs)
                x2 = 2.0 * _mm(bds[s], x1) - hs
                x1s.append(x1)
                x2s.append(x2)
            xc = jnp.concatenate(
                [h, jnp.concatenate(x1s, 0), jnp.concatenate(x2s, 0)], -1)
            y = _mm(xc, w) + b
            return jnp.maximum(y, 0.0) if relu else y

        h = jnp.maximum(_mm(xc_ref[band], wi_ref[0, band]) + bi_ref[0, band],
                        0.0)
        for l in range(nh):
            h = cheb(h, wh_ref[0, band * nh + l], bh_ref[0, band * nh + l],
                     True)
        h = cheb(h, wo_ref[0, band], bo_ref[0, band], False)
        band_outs.append(h)

    g = jnp.concatenate(band_outs, axis=-1)
    taps = []
    for k in range(3):
        p = g * wtap_ref[0, k]
        taps.append(jnp.sum(p.reshape(TB, N, g.shape[-1]), axis=1))
    feats = sum(taps[k][:, k:k + lout] for k in range(3)) + bconv_ref[0]
    o_ref[0] = feats


def _head_kernel(f_ref, gamma_ref, beta_ref, fcw_ref, fcb_ref, o_ref):
    feats = f_ref[0]
    mean = jnp.mean(feats, axis=0, keepdims=True)
    var = jnp.mean((feats - mean) ** 2, axis=0, keepdims=True)
    xn = (feats - mean) * jax.lax.rsqrt(var + 1e-5)
    xn = xn * gamma_ref[0] + beta_ref[0]
    o_ref[0] = (jnp.dot(xn, fcw_ref[0], preferred_element_type=jnp.float32)
                + fcb_ref[0])


def kernel(x, A, wi, bi, wh, bh, wo, bo, wtap, bconv, gamma, beta, fcw, fcb):
    B, _, fin, nbands = x.shape
    M = wi.shape[0]
    H = wi.shape[-1]
    nh2 = wh.shape[1]
    co = wo.shape[-1]
    lout = gamma.shape[-1]
    C = fcw.shape[-1]
    nbt = B // TB

    xb = jnp.transpose(x, (3, 0, 1, 2)).reshape(nbands, B * N, fin)
    ab = jnp.transpose(A, (1, 0, 2, 3)).reshape(nbands, B * N, N)
    wtap_t = jnp.tile(wtap, (1, 1, TB, 1))

    xc1 = pl.pallas_call(
        _cheb1_kernel,
        out_shape=jax.ShapeDtypeStruct((nbands, B * N, 3 * fin), jnp.float32),
        grid=(nbt,),
        in_specs=[
            pl.BlockSpec((nbands, TB * N, fin), lambda i: (0, i, 0)),
            pl.BlockSpec((nbands, TB * N, N), lambda i: (0, i, 0)),
        ],
        out_specs=pl.BlockSpec((nbands, TB * N, 3 * fin), lambda i: (0, i, 0)),
        compiler_params=pltpu.CompilerParams(
            dimension_semantics=("parallel",)),
    )(xb, ab)

    feats = pl.pallas_call(
        _stack_kernel,
        out_shape=jax.ShapeDtypeStruct((M, B, lout), jnp.float32),
        grid=(nbt, M),
        in_specs=[
            pl.BlockSpec((nbands, TB * N, 3 * fin), lambda bt, m: (0, bt, 0)),
            pl.BlockSpec((nbands, TB * N, N), lambda bt, m: (0, bt, 0)),
            pl.BlockSpec((1, nbands, 3 * fin, H), lambda bt, m: (m, 0, 0, 0)),
            pl.BlockSpec((1, nbands, 1, H), lambda bt, m: (m, 0, 0, 0)),
            pl.BlockSpec((1, nh2, 3 * H, H), lambda bt, m: (m, 0, 0, 0)),
            pl.BlockSpec((1, nh2, 1, H), lambda bt, m: (m, 0, 0, 0)),
            pl.BlockSpec((1, nbands, 3 * H, co), lambda bt, m: (m, 0, 0, 0)),
            pl.BlockSpec((1, nbands, 1, co), lambda bt, m: (m, 0, 0, 0)),
            pl.BlockSpec((1, 3, TB * N, 1), lambda bt, m: (m, 0, 0, 0)),
            pl.BlockSpec((1, 1, 1), lambda bt, m: (m, 0, 0)),
        ],
        out_specs=pl.BlockSpec((1, TB, lout), lambda bt, m: (m, bt, 0)),
        compiler_params=pltpu.CompilerParams(
            dimension_semantics=("parallel", "arbitrary")),
    )(xc1, ab, wi, bi, wh, bh, wo, bo, wtap_t, bconv)

    logits = pl.pallas_call(
        _head_kernel,
        out_shape=jax.ShapeDtypeStruct((M, B, C), jnp.float32),
        grid=(M,),
        in_specs=[
            pl.BlockSpec((1, B, lout), lambda m: (m, 0, 0)),
            pl.BlockSpec((1, 1, lout), lambda m: (m, 0, 0)),
            pl.BlockSpec((1, 1, lout), lambda m: (m, 0, 0)),
            pl.BlockSpec((1, lout, C), lambda m: (m, 0, 0)),
            pl.BlockSpec((1, 1, C), lambda m: (m, 0, 0)),
        ],
        out_specs=pl.BlockSpec((1, B, C), lambda m: (m, 0, 0)),
        compiler_params=pltpu.CompilerParams(
            dimension_semantics=("parallel",)),
    )(feats, gamma, beta, fcw, fcb)

    return jnp.mean(logits, axis=0)


# speedup vs baseline: 1.3858x; 1.1496x over previous
import jax
import jax.numpy as jnp
from jax.experimental import pallas as pl
from jax.experimental.pallas import tpu as pltpu

N = 19
TB = 32
TBL = 8


def _mm(a, b):
    return jnp.dot(a, b, preferred_element_type=jnp.float32)


def _blockdiag(lap_stack):
    tiled = jnp.concatenate([lap_stack] * TBL, axis=1)
    row_g = jax.lax.broadcasted_iota(jnp.int32, tiled.shape, 0) // N
    col_g = jax.lax.broadcasted_iota(jnp.int32, tiled.shape, 1) // N
    return jnp.where(row_g == col_g, tiled, 0.0)


def _cheb1_kernel(x_ref, a_ref, o_ref, bd_ref):
    nbands = x_ref.shape[0]
    nsub = TB // TBL
    for band in range(nbands):
        x0 = x_ref[band]
        a = a_ref[band]
        x1s, x2s, bds = [], [], []
        for s in range(nsub):
            bd = _blockdiag(a[s * TBL * N:(s + 1) * TBL * N])
            x0s = x0[s * TBL * N:(s + 1) * TBL * N]
            x1 = _mm(bd, x0s)
            x2 = 2.0 * _mm(bd, x1) - x0s
            x1s.append(x1)
            x2s.append(x2)
            bds.append(bd)
        xc = jnp.concatenate(
            [x0, jnp.concatenate(x1s, 0), jnp.concatenate(x2s, 0)], axis=-1)
        o_ref[band] = xc
        bd_ref[band] = jnp.concatenate(bds, 0)


def _stack_kernel(xc_ref, bd_ref, wi_ref, bi_ref, wh_ref, bh_ref, wo_ref,
                  bo_ref, sel_ref, bconv_ref, o_ref):
    nbands = xc_ref.shape[0]
    nh = wh_ref.shape[1] // nbands
    nsub = TB // TBL
    lout = o_ref.shape[-1]

    band_outs = []
    for band in range(nbands):
        bdb = bd_ref[band]
        bds = [bdb[s * TBL * N:(s + 1) * TBL * N] for s in range(nsub)]

        def cheb(h, w, b, relu):
            x1s, x2s = [], []
            for s in range(nsub):
                hs = h[s * TBL * N:(s + 1) * TBL * N]
                x1 = _mm(bds[s], hs)
                x2 = 2.0 * _mm(bds[s], x1) - hs
                x1s.append(x1)
                x2s.append(x2)
            xc = jnp.concatenate(
                [h, jnp.concatenate(x1s, 0), jnp.concatenate(x2s, 0)], -1)
            y = _mm(xc, w) + b
            return jnp.maximum(y, 0.0) if relu else y

        h = jnp.maximum(_mm(xc_ref[band], wi_ref[0, band]) + bi_ref[0, band],
                        0.0)
        for l in range(nh):
            h = cheb(h, wh_ref[0, band * nh + l], bh_ref[0, band * nh + l],
                     True)
        h = cheb(h, wo_ref[0, band], bo_ref[0, band], False)
        band_outs.append(h)

    g = jnp.concatenate(band_outs, axis=-1)
    taps = [_mm(sel_ref[0, k], g) for k in range(3)]
    feats = sum(taps[k][:, k:k + lout] for k in range(3)) + bconv_ref[0]
    o_ref[0] = feats


def _head_kernel(f_ref, gamma_ref, beta_ref, fcw_ref, fcb_ref, o_ref):
    feats = f_ref[0]
    mean = jnp.mean(feats, axis=0, keepdims=True)
    var = jnp.mean((feats - mean) ** 2, axis=0, keepdims=True)
    xn = (feats - mean) * jax.lax.rsqrt(var + 1e-5)
    xn = xn * gamma_ref[0] + beta_ref[0]
    o_ref[0] = (jnp.dot(xn, fcw_ref[0], preferred_element_type=jnp.float32)
                + fcb_ref[0])


def kernel(x, A, wi, bi, wh, bh, wo, bo, wtap, bconv, gamma, beta, fcw, fcb):
    B, _, fin, nbands = x.shape
    M = wi.shape[0]
    H = wi.shape[-1]
    nh2 = wh.shape[1]
    co = wo.shape[-1]
    lout = gamma.shape[-1]
    C = fcw.shape[-1]
    nbt = B // TB

    xb = jnp.transpose(x, (3, 0, 1, 2)).reshape(nbands, B * N, fin)
    ab = jnp.transpose(A, (1, 0, 2, 3)).reshape(nbands, B * N, N)
    eye = jnp.eye(TB, dtype=jnp.float32)
    sel = (eye[None, None, :, :, None]
           * wtap[:, :, None, None, :, 0]).reshape(M, 3, TB, TB * N)

    xc1, bdl = pl.pallas_call(
        _cheb1_kernel,
        out_shape=(
            jax.ShapeDtypeStruct((nbands, B * N, 3 * fin), jnp.float32),
            jax.ShapeDtypeStruct((nbands, B * N, TBL * N), jnp.float32),
        ),
        grid=(nbt,),
        in_specs=[
            pl.BlockSpec((nbands, TB * N, fin), lambda i: (0, i, 0)),
            pl.BlockSpec((nbands, TB * N, N), lambda i: (0, i, 0)),
        ],
        out_specs=(
            pl.BlockSpec((nbands, TB * N, 3 * fin), lambda i: (0, i, 0)),
            pl.BlockSpec((nbands, TB * N, TBL * N), lambda i: (0, i, 0)),
        ),
        compiler_params=pltpu.CompilerParams(
            dimension_semantics=("parallel",)),
    )(xb, ab)

    feats = pl.pallas_call(
        _stack_kernel,
        out_shape=jax.ShapeDtypeStruct((M, B, lout), jnp.float32),
        grid=(nbt, M),
        in_specs=[
            pl.BlockSpec((nbands, TB * N, 3 * fin), lambda bt, m: (0, bt, 0)),
            pl.BlockSpec((nbands, TB * N, TBL * N), lambda bt, m: (0, bt, 0)),
            pl.BlockSpec((1, nbands, 3 * fin, H), lambda bt, m: (m, 0, 0, 0)),
            pl.BlockSpec((1, nbands, 1, H), lambda bt, m: (m, 0, 0, 0)),
            pl.BlockSpec((1, nh2, 3 * H, H), lambda bt, m: (m, 0, 0, 0)),
            pl.BlockSpec((1, nh2, 1, H), lambda bt, m: (m, 0, 0, 0)),
            pl.BlockSpec((1, nbands, 3 * H, co), lambda bt, m: (m, 0, 0, 0)),
            pl.BlockSpec((1, nbands, 1, co), lambda bt, m: (m, 0, 0, 0)),
            pl.BlockSpec((1, 3, TB, TB * N), lambda bt, m: (m, 0, 0, 0)),
            pl.BlockSpec((1, 1, 1), lambda bt, m: (m, 0, 0)),
        ],
        out_specs=pl.BlockSpec((1, TB, lout), lambda bt, m: (m, bt, 0)),
        compiler_params=pltpu.CompilerParams(
            dimension_semantics=("parallel", "arbitrary")),
    )(xc1, bdl, wi, bi, wh, bh, wo, bo, sel, bconv)

    logits = pl.pallas_call(
        _head_kernel,
        out_shape=jax.ShapeDtypeStruct((M, B, C), jnp.float32),
        grid=(M,),
        in_specs=[
            pl.BlockSpec((1, B, lout), lambda m: (m, 0, 0)),
            pl.BlockSpec((1, 1, lout), lambda m: (m, 0, 0)),
            pl.BlockSpec((1, 1, lout), lambda m: (m, 0, 0)),
            pl.BlockSpec((1, lout, C), lambda m: (m, 0, 0)),
            pl.BlockSpec((1, 1, C), lambda m: (m, 0, 0)),
        ],
        out_specs=pl.BlockSpec((1, B, C), lambda m: (m, 0, 0)),
        compiler_params=pltpu.CompilerParams(
            dimension_semantics=("parallel",)),
    )(feats, gamma, beta, fcw, fcb)

    return jnp.mean(logits, axis=0)
